# baseline (speedup 1.0000x reference)
import functools

import jax
import jax.numpy as jnp
from jax import lax
from jax.experimental import pallas as pl
from jax.experimental.pallas import tpu as pltpu
from jax.experimental.pallas import tpu_sc as plsc

MAXD = 20
NBUCKET = 2 * MAXD + 1


@functools.lru_cache(maxsize=None)
def _build_sc_call(B, N, H):
    L = 16
    NC, NS = 2, 16
    NW = NC * NS
    ROWS = B * N
    assert ROWS % (2 * NW) == 0
    RPW = ROWS // NW
    assert N % RPW == 0
    WPB = N // RPW
    CH = N // L
    TPH = 2 * NBUCKET
    TFLAT = H * 2 * TPH
    assert TFLAT % L == 0

    mesh = plsc.VectorSubcoreMesh(core_axis_name="c", subcore_axis_name="s")

    @functools.partial(
        pl.kernel,
        mesh=mesh,
        out_type=jax.ShapeDtypeStruct((B * H * N * N,), jnp.float32),
        compiler_params=pltpu.CompilerParams(needs_layout_passes=False),
        scratch_types=[
            pltpu.VMEM((N,), jnp.float32),
            pltpu.VMEM((N,), jnp.int32),
            pltpu.VMEM((RPW,), jnp.float32),
            pltpu.VMEM((RPW,), jnp.int32),
            pltpu.VMEM((NBUCKET * H,), jnp.float32),
            pltpu.VMEM((H * 4,), jnp.float32),
            pltpu.VMEM((TFLAT,), jnp.float32),
            pltpu.VMEM((H * N,), jnp.float32),
            pltpu.VMEM((H * N,), jnp.float32),
            pltpu.SemaphoreType.DMA,
            pltpu.SemaphoreType.DMA,
        ],
    )
    def sc_call(ci_hbm, cj_hbm, ti_hbm, tj_hbm, rel_hbm, tt_hbm, out_hbm,
                cj_v, tj_v, ci_v, ti_v, rel_v, tt_v, tab_v,
                rows0, rows1, sem0, sem1):
        c = lax.axis_index("c")
        s = lax.axis_index("s")
        wid = s * NC + c
        b = wid // WPB
        i_base = (wid % WPB) * RPW
        out_base = b * (H * N * N) + i_base * N

        pltpu.sync_copy(cj_hbm.at[b], cj_v)
        pltpu.sync_copy(tj_hbm.at[b], tj_v)
        pltpu.sync_copy(ci_hbm.at[b, pl.ds(i_base, RPW)], ci_v)
        pltpu.sync_copy(ti_hbm.at[b, pl.ds(i_base, RPW)], ti_v)
        pltpu.sync_copy(rel_hbm, rel_v)
        pltpu.sync_copy(tt_hbm, tt_v)

        def build_body(g, carry):
            off = pl.multiple_of(g * L, L)
            f = off + lax.iota(jnp.int32, L)
            h = f // (2 * TPH)
            r = f - h * (2 * TPH)
            t = r // TPH
            q = r - t * TPH
            k = q >> 1
            m = q & 1
            a = plsc.load_gather(rel_v, [k * H + h])
            bb = plsc.load_gather(tt_v, [h * 4 + t * 2 + m])
            tab_v[pl.ds(off, L)] = a + bb
            return carry

        lax.fori_loop(0, TFLAT // L, build_body, 0)

        def compute_row(r, rows_ref):
            rsplat = jnp.full((L,), r, jnp.int32)
            ci_s = plsc.load_gather(ci_v, [rsplat])
            base0 = plsc.load_gather(ti_v, [rsplat]) * TPH

            @plsc.parallel_loop(0, CH, unroll=4)
            def _chunk(g):
                off = pl.multiple_of(g * L, L)
                d = ci_s - cj_v[pl.ds(off, L)]
                d = jnp.minimum(jnp.maximum(d, -20.0), 20.0)
                z = (d + 20.0) + 0.5
                idx16 = z.astype(jnp.int32)
                tie = (idx16.astype(jnp.float32) == z) & ((idx16 & 1) == 1)
                idx16 = idx16 - tie.astype(jnp.int32)
                cidx = idx16 * 2 + tj_v[pl.ds(off, L)] + base0
                vals = [plsc.load_gather(tab_v, [cidx + h * (2 * TPH)])
                        for h in range(H)]
                for h in range(H):
                    rows_ref[pl.ds(h * N + off, L)] = vals[h]

        def issue_row(r, rows_ref, sem):
            base = out_base + r * N
            for h in range(H):
                pltpu.async_copy(
                    rows_ref.at[pl.ds(h * N, N)],
                    out_hbm.at[pl.ds(base + h * (N * N), N)],
                    sem,
                )

        def drain(rows_ref, sem):
            pltpu.make_async_copy(out_hbm.at[pl.ds(0, H * N)], rows_ref,
                                  sem).wait()

        def pair_body(p, carry):
            @pl.when(p > 0)
            def _():
                drain(rows0, sem0)

            compute_row(2 * p, rows0)
            issue_row(2 * p, rows0, sem0)

            @pl.when(p > 0)
            def _():
                drain(rows1, sem1)

            compute_row(2 * p + 1, rows1)
            issue_row(2 * p + 1, rows1, sem1)
            return carry

        lax.fori_loop(0, RPW // 2, pair_body, 0)
        drain(rows0, sem0)
        drain(rows1, sem1)

    return sc_call


def kernel(centers_i, centers_j, types_i, types_j, rel, tt):
    B, N = centers_i.shape
    H = rel.shape[1]
    call = _build_sc_call(B, N, H)
    out = call(
        centers_i.astype(jnp.float32),
        centers_j.astype(jnp.float32),
        types_i.astype(jnp.int32),
        types_j.astype(jnp.int32),
        rel.reshape(-1).astype(jnp.float32),
        tt.reshape(-1).astype(jnp.float32),
    )
    return out.reshape(B, H, N, N)


# speedup vs baseline: 3.2536x; 3.2536x over previous
import functools

import jax
import jax.numpy as jnp
from jax import lax
from jax.experimental import pallas as pl
from jax.experimental.pallas import tpu as pltpu
from jax.experimental.pallas import tpu_sc as plsc

MAXD = 20
NBUCKET = 2 * MAXD + 1


@functools.lru_cache(maxsize=None)
def _build_sc_call(B, N, H):
    L = 16
    NC, NS = 2, 16
    NW = NC * NS
    ROWS = B * N
    assert ROWS % (2 * NW) == 0
    RPW = ROWS // NW
    assert N % RPW == 0
    WPB = N // RPW
    CH = N // L
    TPH = 2 * NBUCKET
    TFLAT = H * 2 * TPH
    assert TFLAT % L == 0

    mesh = plsc.VectorSubcoreMesh(core_axis_name="c", subcore_axis_name="s")

    @functools.partial(
        pl.kernel,
        mesh=mesh,
        out_type=jax.ShapeDtypeStruct((B, H, N, N), jnp.float32),
        compiler_params=pltpu.CompilerParams(needs_layout_passes=False),
        scratch_types=[
            pltpu.VMEM((N,), jnp.float32),
            pltpu.VMEM((N,), jnp.int32),
            pltpu.VMEM((RPW,), jnp.float32),
            pltpu.VMEM((RPW,), jnp.int32),
            pltpu.VMEM((NBUCKET * H,), jnp.float32),
            pltpu.VMEM((H * 4,), jnp.float32),
            pltpu.VMEM((TFLAT,), jnp.float32),
            pltpu.VMEM((H * N,), jnp.float32),
            pltpu.VMEM((H * N,), jnp.float32),
            pltpu.SemaphoreType.DMA,
            pltpu.SemaphoreType.DMA,
        ],
    )
    def sc_call(ci_hbm, cj_hbm, ti_hbm, tj_hbm, rel_hbm, tt_hbm, out_hbm,
                cj_v, tj_v, ci_v, ti_v, rel_v, tt_v, tab_v,
                rows0, rows1, sem0, sem1):
        c = lax.axis_index("c")
        s = lax.axis_index("s")
        wid = s * NC + c
        b = wid // WPB
        i_base = (wid % WPB) * RPW

        pltpu.sync_copy(cj_hbm.at[b], cj_v)
        pltpu.sync_copy(tj_hbm.at[b], tj_v)
        pltpu.sync_copy(ci_hbm.at[b, pl.ds(i_base, RPW)], ci_v)
        pltpu.sync_copy(ti_hbm.at[b, pl.ds(i_base, RPW)], ti_v)
        pltpu.sync_copy(rel_hbm, rel_v)
        pltpu.sync_copy(tt_hbm, tt_v)

        def build_body(g, carry):
            off = pl.multiple_of(g * L, L)
            f = off + lax.iota(jnp.int32, L)
            h = f // (2 * TPH)
            r = f - h * (2 * TPH)
            t = r // TPH
            q = r - t * TPH
            k = q >> 1
            m = q & 1
            a = plsc.load_gather(rel_v, [k * H + h])
            bb = plsc.load_gather(tt_v, [h * 4 + t * 2 + m])
            tab_v[pl.ds(off, L)] = a + bb
            return carry

        lax.fori_loop(0, TFLAT // L, build_body, 0)

        def compute_row(r, rows_ref):
            rsplat = jnp.full((L,), r, jnp.int32)
            ci_s = plsc.load_gather(ci_v, [rsplat])
            base0 = plsc.load_gather(ti_v, [rsplat]) * TPH

            @plsc.parallel_loop(0, CH, unroll=2)
            def _chunk(g):
                off = pl.multiple_of(g * L, L)
                d = ci_s - cj_v[pl.ds(off, L)]
                d = jnp.minimum(jnp.maximum(d, -20.0), 20.0)
                z = (d + 20.0) + 0.5
                idx16 = z.astype(jnp.int32)
                tie = (idx16.astype(jnp.float32) == z) & ((idx16 & 1) == 1)
                idx16 = idx16 - tie.astype(jnp.int32)
                cidx = idx16 * 2 + tj_v[pl.ds(off, L)] + base0
                vals = [plsc.load_gather(tab_v, [cidx + h * (2 * TPH)])
                        for h in range(H)]
                for h in range(H):
                    rows_ref[pl.ds(h * N + off, L)] = vals[h]

        def issue_row(r, rows_ref, sem):
            i = i_base + r
            for h in range(H):
                pltpu.async_copy(rows_ref.at[pl.ds(h * N, N)],
                                 out_hbm.at[b, h, i], sem)

        def drain_row(r, rows_ref, sem):
            i = i_base + r
            for h in range(H):
                pltpu.make_async_copy(rows_ref.at[pl.ds(h * N, N)],
                                      out_hbm.at[b, h, i], sem).wait()

        def pair_body(p, carry):
            @pl.when(p > 0)
            def _():
                drain_row(2 * p - 2, rows0, sem0)

            compute_row(2 * p, rows0)
            issue_row(2 * p, rows0, sem0)

            @pl.when(p > 0)
            def _():
                drain_row(2 * p - 1, rows1, sem1)

            compute_row(2 * p + 1, rows1)
            issue_row(2 * p + 1, rows1, sem1)
            return carry

        lax.fori_loop(0, RPW // 2, pair_body, 0)
        drain_row(RPW - 2, rows0, sem0)
        drain_row(RPW - 1, rows1, sem1)

    return sc_call


def kernel(centers_i, centers_j, types_i, types_j, rel, tt):
    B, N = centers_i.shape
    H = rel.shape[1]
    call = _build_sc_call(B, N, H)
    return call(
        centers_i.astype(jnp.float32),
        centers_j.astype(jnp.float32),
        types_i.astype(jnp.int32),
        types_j.astype(jnp.int32),
        rel.reshape(-1).astype(jnp.float32),
        tt.reshape(-1).astype(jnp.float32),
    )
